# initial kernel scaffold (unmeasured)
import jax
import jax.numpy as jnp
from jax import lax
from jax.experimental import pallas as pl
from jax.experimental.pallas import tpu as pltpu

N_DEV = 8


def kernel(x, w_mat, scale_x, scale_w):
    M, Ks = x.shape
    N = w_mat.shape[1]
    m_per = M // N_DEV

    x8 = x.astype(jnp.float8_e4m3fn)
    w8 = w_mat.astype(jnp.float8_e4m3fn)
    s = (scale_x * scale_w).reshape(1, 1)

    def dotf(a, b):
        return lax.dot_general(
            a, b, (((1,), (0,)), ((), ())),
            preferred_element_type=jnp.float32,
        )

    def body(x_ref, w_ref, s_ref, out_ref, xp_ref, wbuf_ref,
             x_send_sems, x_recv_sems, w_send_sems, w_recv_sems):
        me = lax.axis_index("i")
        right = lax.rem(me + 1, N_DEV)

        barrier = pltpu.get_barrier_semaphore()
        for d in range(1, N_DEV):
            peer = lax.rem(me + d, N_DEV)
            pl.semaphore_signal(
                barrier, inc=1,
                device_id=(peer,), device_id_type=pl.DeviceIdType.MESH,
            )
        pl.semaphore_wait(barrier, N_DEV - 1)

        x_rdmas = []
        for d in range(1, N_DEV):
            dest = lax.rem(me + d, N_DEV)
            rd = pltpu.make_async_remote_copy(
                src_ref=x_ref.at[pl.ds(dest * m_per, m_per), :],
                dst_ref=xp_ref.at[d],
                send_sem=x_send_sems.at[d],
                recv_sem=x_recv_sems.at[d],
                device_id=(dest,),
                device_id_type=pl.DeviceIdType.MESH,
            )
            rd.start()
            x_rdmas.append(rd)

        w_rdmas = []
        rd0 = pltpu.make_async_remote_copy(
            src_ref=w_ref,
            dst_ref=wbuf_ref.at[1],
            send_sem=w_send_sems.at[0],
            recv_sem=w_recv_sems.at[1],
            device_id=(right,),
            device_id_type=pl.DeviceIdType.MESH,
        )
        rd0.start()
        w_rdmas.append(rd0)

        xo = x_ref[pl.ds(me * m_per, m_per), :]
        out_ref[...] = dotf(xo, w_ref[...])

        for h in range(1, N_DEV):
            recv = pltpu.make_async_remote_copy(
                src_ref=wbuf_ref.at[h],
                dst_ref=wbuf_ref.at[h],
                send_sem=w_send_sems.at[h],
                recv_sem=w_recv_sems.at[h],
                device_id=(right,),
                device_id_type=pl.DeviceIdType.MESH,
            )
            recv.wait_recv()
            if h < N_DEV - 1:
                rd = pltpu.make_async_remote_copy(
                    src_ref=wbuf_ref.at[h],
                    dst_ref=wbuf_ref.at[h + 1],
                    send_sem=w_send_sems.at[h],
                    recv_sem=w_recv_sems.at[h + 1],
                    device_id=(right,),
                    device_id_type=pl.DeviceIdType.MESH,
                )
                rd.start()
                w_rdmas.append(rd)

            xrecv = pltpu.make_async_remote_copy(
                src_ref=xp_ref.at[h],
                dst_ref=xp_ref.at[h],
                send_sem=x_send_sems.at[h],
                recv_sem=x_recv_sems.at[h],
                device_id=(right,),
                device_id_type=pl.DeviceIdType.MESH,
            )
            xrecv.wait_recv()
            out_ref[...] = out_ref[...] + dotf(xp_ref[h], wbuf_ref[h])

        out_ref[...] = out_ref[...] * s_ref[0, 0]

        for rd in x_rdmas + w_rdmas:
            rd.wait_send()

    return pl.pallas_call(
        body,
        out_shape=jax.ShapeDtypeStruct((m_per, N), jnp.float32),
        in_specs=[
            pl.BlockSpec(memory_space=pltpu.VMEM),
            pl.BlockSpec(memory_space=pltpu.VMEM),
            pl.BlockSpec(memory_space=pltpu.SMEM),
        ],
        out_specs=pl.BlockSpec(memory_space=pltpu.VMEM),
        scratch_shapes=[
            pltpu.VMEM((N_DEV, m_per, Ks), jnp.float8_e4m3fn),
            pltpu.VMEM((N_DEV, Ks, N), jnp.float8_e4m3fn),
            pltpu.SemaphoreType.DMA((N_DEV,)),
            pltpu.SemaphoreType.DMA((N_DEV,)),
            pltpu.SemaphoreType.DMA((N_DEV,)),
            pltpu.SemaphoreType.DMA((N_DEV,)),
        ],
        compiler_params=pltpu.CompilerParams(collective_id=0),
    )(x8, w8, s)


# baseline (device time: 382120 ns/iter reference)
import jax
import jax.numpy as jnp
from jax import lax
from jax.experimental import pallas as pl
from jax.experimental.pallas import tpu as pltpu

N_DEV = 8


def kernel(x, w_mat, scale_x, scale_w):
    M, Ks = x.shape
    N = w_mat.shape[1]
    m_per = M // N_DEV

    x8 = x.astype(jnp.float8_e4m3fn)
    w8 = w_mat.astype(jnp.float8_e4m3fn)
    s = (scale_x * scale_w).reshape(1, 1)

    def dotf(a, b):
        return lax.dot_general(
            a, b, (((1,), (0,)), ((), ())),
            preferred_element_type=jnp.float32,
        )

    def body(x_ref, w_ref, s_ref, out_ref, xp_ref, wbuf_ref,
             x_send_sems, x_recv_sems, w_send_sems, w_recv_sems):
        me = lax.axis_index("i")
        right = lax.rem(me + 1, N_DEV)

        barrier = pltpu.get_barrier_semaphore()
        for d in range(1, N_DEV):
            peer = lax.rem(me + d, N_DEV)
            pl.semaphore_signal(
                barrier, inc=1,
                device_id=(peer,), device_id_type=pl.DeviceIdType.MESH,
            )
        pl.semaphore_wait(barrier, N_DEV - 1)

        x_rdmas = []
        for d in range(1, N_DEV):
            dest = lax.rem(me + d, N_DEV)
            rd = pltpu.make_async_remote_copy(
                src_ref=x_ref.at[pl.ds(dest * m_per, m_per), :],
                dst_ref=xp_ref.at[d],
                send_sem=x_send_sems.at[d],
                recv_sem=x_recv_sems.at[d],
                device_id=(dest,),
                device_id_type=pl.DeviceIdType.MESH,
            )
            rd.start()
            x_rdmas.append(rd)

        w_rdmas = []
        rd0 = pltpu.make_async_remote_copy(
            src_ref=w_ref,
            dst_ref=wbuf_ref.at[1],
            send_sem=w_send_sems.at[0],
            recv_sem=w_recv_sems.at[1],
            device_id=(right,),
            device_id_type=pl.DeviceIdType.MESH,
        )
        rd0.start()
        w_rdmas.append(rd0)

        xo = x_ref[pl.ds(me * m_per, m_per), :]
        out_ref[...] = dotf(xo, w_ref[...])

        for h in range(1, N_DEV):
            recv = pltpu.make_async_remote_copy(
                src_ref=wbuf_ref.at[h],
                dst_ref=wbuf_ref.at[h],
                send_sem=w_send_sems.at[h],
                recv_sem=w_recv_sems.at[h],
                device_id=(right,),
                device_id_type=pl.DeviceIdType.MESH,
            )
            recv.wait_recv()
            if h < N_DEV - 1:
                rd = pltpu.make_async_remote_copy(
                    src_ref=wbuf_ref.at[h],
                    dst_ref=wbuf_ref.at[h + 1],
                    send_sem=w_send_sems.at[h],
                    recv_sem=w_recv_sems.at[h + 1],
                    device_id=(right,),
                    device_id_type=pl.DeviceIdType.MESH,
                )
                rd.start()
                w_rdmas.append(rd)

            xrecv = pltpu.make_async_remote_copy(
                src_ref=xp_ref.at[h],
                dst_ref=xp_ref.at[h],
                send_sem=x_send_sems.at[h],
                recv_sem=x_recv_sems.at[h],
                device_id=(right,),
                device_id_type=pl.DeviceIdType.MESH,
            )
            xrecv.wait_recv()
            out_ref[...] = out_ref[...] + dotf(xp_ref[h], wbuf_ref[h])

        out_ref[...] = out_ref[...] * s_ref[0, 0]

        for rd in x_rdmas + w_rdmas:
            rd.wait_send()

    return pl.pallas_call(
        body,
        out_shape=jax.ShapeDtypeStruct((m_per, N), jnp.float32),
        in_specs=[
            pl.BlockSpec(memory_space=pltpu.VMEM),
            pl.BlockSpec(memory_space=pltpu.VMEM),
            pl.BlockSpec(memory_space=pltpu.SMEM),
        ],
        out_specs=pl.BlockSpec(memory_space=pltpu.VMEM),
        scratch_shapes=[
            pltpu.VMEM((N_DEV, m_per, Ks), jnp.float8_e4m3fn),
            pltpu.VMEM((N_DEV, Ks, N), jnp.float8_e4m3fn),
            pltpu.SemaphoreType.DMA((N_DEV,)),
            pltpu.SemaphoreType.DMA((N_DEV,)),
            pltpu.SemaphoreType.DMA((N_DEV,)),
            pltpu.SemaphoreType.DMA((N_DEV,)),
        ],
        compiler_params=pltpu.CompilerParams(
            collective_id=0,
            vmem_limit_bytes=100 * 1024 * 1024,
        ),
    )(x8, w8, s)


# device time: 370849 ns/iter; 1.0304x vs baseline; 1.0304x over previous
import jax
import jax.numpy as jnp
from jax import lax
from jax.experimental import pallas as pl
from jax.experimental.pallas import tpu as pltpu

N_DEV = 8


def kernel(x, w_mat, scale_x, scale_w):
    M, Ks = x.shape
    N = w_mat.shape[1]
    m_per = M // N_DEV

    x8 = x.astype(jnp.float8_e4m3fn)
    w8 = w_mat.astype(jnp.float8_e4m3fn)
    s = (scale_x * scale_w).reshape(1, 1)

    def dotf(a, b):
        return lax.dot_general(
            a, b, (((1,), (0,)), ((), ())),
            preferred_element_type=jnp.float32,
        )

    def body(x_ref, w_ref, s_ref, out_ref, xp_ref, wbuf_ref,
             x_send_sems, x_recv_sems, w_send_sems, w_recv_sems):
        me = lax.axis_index("i")
        right = lax.rem(me + 1, N_DEV)

        barrier = pltpu.get_barrier_semaphore()
        for d in range(1, N_DEV):
            peer = lax.rem(me + d, N_DEV)
            pl.semaphore_signal(
                barrier, inc=1,
                device_id=(peer,), device_id_type=pl.DeviceIdType.MESH,
            )
        pl.semaphore_wait(barrier, N_DEV - 1)

        x_rdmas = []
        for d in range(1, N_DEV):
            dest = lax.rem(me + d, N_DEV)
            rd = pltpu.make_async_remote_copy(
                src_ref=x_ref.at[pl.ds(dest * m_per, m_per), :],
                dst_ref=xp_ref.at[d],
                send_sem=x_send_sems.at[d],
                recv_sem=x_recv_sems.at[d],
                device_id=(dest,),
                device_id_type=pl.DeviceIdType.MESH,
            )
            rd.start()
            x_rdmas.append(rd)

        NPC = 2
        n_piece = N // NPC

        def w_copy(src_slot, dst_slot, p):
            return pltpu.make_async_remote_copy(
                src_ref=wbuf_ref.at[src_slot, :, pl.ds(p * n_piece, n_piece)],
                dst_ref=wbuf_ref.at[dst_slot, :, pl.ds(p * n_piece, n_piece)],
                send_sem=w_send_sems.at[src_slot, p],
                recv_sem=w_recv_sems.at[dst_slot, p],
                device_id=(right,),
                device_id_type=pl.DeviceIdType.MESH,
            )

        w_rdmas = []
        for p in range(NPC):
            rd = pltpu.make_async_remote_copy(
                src_ref=w_ref.at[:, pl.ds(p * n_piece, n_piece)],
                dst_ref=wbuf_ref.at[1, :, pl.ds(p * n_piece, n_piece)],
                send_sem=w_send_sems.at[0, p],
                recv_sem=w_recv_sems.at[1, p],
                device_id=(right,),
                device_id_type=pl.DeviceIdType.MESH,
            )
            rd.start()
            w_rdmas.append(rd)

        xo = x_ref[pl.ds(me * m_per, m_per), :]
        out_ref[...] = dotf(xo, w_ref[...])

        for h in range(1, N_DEV):
            xrecv = pltpu.make_async_remote_copy(
                src_ref=xp_ref.at[h],
                dst_ref=xp_ref.at[h],
                send_sem=x_send_sems.at[h],
                recv_sem=x_recv_sems.at[h],
                device_id=(right,),
                device_id_type=pl.DeviceIdType.MESH,
            )
            for p in range(NPC):
                recv = w_copy(h, h, p)
                recv.wait_recv()
                if h < N_DEV - 1:
                    rd = w_copy(h, h + 1, p)
                    rd.start()
                    w_rdmas.append(rd)
                if p == 0:
                    xrecv.wait_recv()
                cols = pl.ds(p * n_piece, n_piece)
                out_ref[:, cols] = out_ref[:, cols] + dotf(
                    xp_ref[h], wbuf_ref[h, :, cols]
                )

        out_ref[...] = out_ref[...] * s_ref[0, 0]

        for rd in x_rdmas + w_rdmas:
            rd.wait_send()

    return pl.pallas_call(
        body,
        out_shape=jax.ShapeDtypeStruct((m_per, N), jnp.float32),
        in_specs=[
            pl.BlockSpec(memory_space=pltpu.VMEM),
            pl.BlockSpec(memory_space=pltpu.VMEM),
            pl.BlockSpec(memory_space=pltpu.SMEM),
        ],
        out_specs=pl.BlockSpec(memory_space=pltpu.VMEM),
        scratch_shapes=[
            pltpu.VMEM((N_DEV, m_per, Ks), jnp.float8_e4m3fn),
            pltpu.VMEM((N_DEV, Ks, N), jnp.float8_e4m3fn),
            pltpu.SemaphoreType.DMA((N_DEV,)),
            pltpu.SemaphoreType.DMA((N_DEV,)),
            pltpu.SemaphoreType.DMA((N_DEV, 2)),
            pltpu.SemaphoreType.DMA((N_DEV, 2)),
        ],
        compiler_params=pltpu.CompilerParams(
            collective_id=0,
            vmem_limit_bytes=100 * 1024 * 1024,
        ),
    )(x8, w8, s)


# device time: 235787 ns/iter; 1.6206x vs baseline; 1.5728x over previous
import jax
import jax.numpy as jnp
from jax import lax
from jax.experimental import pallas as pl
from jax.experimental.pallas import tpu as pltpu

N_DEV = 8
N_R = 4
N_L = N_DEV - 1 - N_R
NPC = 2


def kernel(x, w_mat, scale_x, scale_w):
    M, Ks = x.shape
    N = w_mat.shape[1]
    m_per = M // N_DEV
    n_piece = N // NPC

    x8 = x.astype(jnp.float8_e4m3fn)
    w8 = w_mat.astype(jnp.float8_e4m3fn)
    s = (scale_x * scale_w).reshape(1, 1)

    def dotf(a, b):
        return lax.dot_general(
            a, b, (((1,), (0,)), ((), ())),
            preferred_element_type=jnp.float32,
        )

    def body(x_ref, w_ref, s_ref, out_ref, xp_ref, wr_ref, wl_ref,
             x_send_sems, x_recv_sems,
             wr_send_sems, wr_recv_sems, wl_send_sems, wl_recv_sems):
        me = lax.axis_index("i")
        right = lax.rem(me + 1, N_DEV)
        left = lax.rem(me + N_DEV - 1, N_DEV)

        barrier = pltpu.get_barrier_semaphore()
        for d in range(1, N_DEV):
            peer = lax.rem(me + d, N_DEV)
            pl.semaphore_signal(
                barrier, inc=1,
                device_id=(peer,), device_id_type=pl.DeviceIdType.MESH,
            )
        pl.semaphore_wait(barrier, N_DEV - 1)

        def w_send(buf, src_slot, dst_slot, p, ssems, rsems, dev):
            src = (w_ref.at[:, pl.ds(p * n_piece, n_piece)] if src_slot == 0
                   else buf.at[src_slot, :, pl.ds(p * n_piece, n_piece)])
            return pltpu.make_async_remote_copy(
                src_ref=src,
                dst_ref=buf.at[dst_slot, :, pl.ds(p * n_piece, n_piece)],
                send_sem=ssems.at[src_slot, p],
                recv_sem=rsems.at[dst_slot, p],
                device_id=(dev,),
                device_id_type=pl.DeviceIdType.MESH,
            )

        rdmas = []

        for p in range(NPC):
            rd = w_send(wr_ref, 0, 1, p, wr_send_sems, wr_recv_sems, right)
            rd.start()
            rdmas.append(rd)
            rd = w_send(wl_ref, 0, 1, p, wl_send_sems, wl_recv_sems, left)
            rd.start()
            rdmas.append(rd)

        for d in range(1, N_DEV):
            dest = lax.rem(me + d, N_DEV)
            rd = pltpu.make_async_remote_copy(
                src_ref=x_ref.at[pl.ds(dest * m_per, m_per), :],
                dst_ref=xp_ref.at[d],
                send_sem=x_send_sems.at[d],
                recv_sem=x_recv_sems.at[d],
                device_id=(dest,),
                device_id_type=pl.DeviceIdType.MESH,
            )
            rd.start()
            rdmas.append(rd)

        xo = x_ref[pl.ds(me * m_per, m_per), :]
        out_ref[...] = dotf(xo, w_ref[...])

        def x_wait(slot):
            pltpu.make_async_remote_copy(
                src_ref=xp_ref.at[slot],
                dst_ref=xp_ref.at[slot],
                send_sem=x_send_sems.at[slot],
                recv_sem=x_recv_sems.at[slot],
                device_id=(right,),
                device_id_type=pl.DeviceIdType.MESH,
            ).wait_recv()

        def stream_step(buf, s, n_hops, ssems, rsems, dev, xp_slot):
            for p in range(NPC):
                w_send(buf, s, s, p, ssems, rsems, dev).wait_recv()
                if s < n_hops:
                    rd = w_send(buf, s, s + 1, p, ssems, rsems, dev)
                    rd.start()
                    rdmas.append(rd)
                if p == 0:
                    x_wait(xp_slot)
                cols = pl.ds(p * n_piece, n_piece)
                out_ref[:, cols] = out_ref[:, cols] + dotf(
                    xp_ref[xp_slot], buf[s, :, cols]
                )

        for st in range(1, N_R + 1):
            stream_step(wr_ref, st, N_R, wr_send_sems, wr_recv_sems,
                        right, st)
            if st <= N_L:
                stream_step(wl_ref, st, N_L, wl_send_sems, wl_recv_sems,
                            left, N_DEV - st)

        out_ref[...] = out_ref[...] * s_ref[0, 0]

        for rd in rdmas:
            rd.wait_send()

    return pl.pallas_call(
        body,
        out_shape=jax.ShapeDtypeStruct((m_per, N), jnp.float32),
        in_specs=[
            pl.BlockSpec(memory_space=pltpu.VMEM),
            pl.BlockSpec(memory_space=pltpu.VMEM),
            pl.BlockSpec(memory_space=pltpu.SMEM),
        ],
        out_specs=pl.BlockSpec(memory_space=pltpu.VMEM),
        scratch_shapes=[
            pltpu.VMEM((N_DEV, m_per, Ks), jnp.float8_e4m3fn),
            pltpu.VMEM((N_R + 1, Ks, N), jnp.float8_e4m3fn),
            pltpu.VMEM((N_L + 1, Ks, N), jnp.float8_e4m3fn),
            pltpu.SemaphoreType.DMA((N_DEV,)),
            pltpu.SemaphoreType.DMA((N_DEV,)),
            pltpu.SemaphoreType.DMA((N_R + 1, NPC)),
            pltpu.SemaphoreType.DMA((N_R + 1, NPC)),
            pltpu.SemaphoreType.DMA((N_L + 1, NPC)),
            pltpu.SemaphoreType.DMA((N_L + 1, NPC)),
        ],
        compiler_params=pltpu.CompilerParams(
            collective_id=0,
            vmem_limit_bytes=100 * 1024 * 1024,
        ),
    )(x8, w8, s)


# device time: 216539 ns/iter; 1.7647x vs baseline; 1.0889x over previous
import jax
import jax.numpy as jnp
from jax import lax
from jax.experimental import pallas as pl
from jax.experimental.pallas import tpu as pltpu

N_DEV = 8
N_HOP = 4
NPC = 2


def kernel(x, w_mat, scale_x, scale_w):
    M, Ks = x.shape
    N = w_mat.shape[1]
    m_per = M // N_DEV
    n_piece = N // NPC

    x8 = x.astype(jnp.float8_e4m3fn)
    w8 = w_mat.astype(jnp.float8_e4m3fn)
    s = (scale_x * scale_w).reshape(1, 1)

    def dotf(a, b):
        return lax.dot_general(
            a, b, (((1,), (0,)), ((), ())),
            preferred_element_type=jnp.float32,
        )

    def body(x_ref, w_ref, s_ref, out_ref, xp_ref, wr_ref, wl_ref,
             x_send_sems, x_recv_sems,
             wr_send_sems, wr_recv_sems, wl_send_sems, wl_recv_sems):
        me = lax.axis_index("i")
        right = lax.rem(me + 1, N_DEV)
        left = lax.rem(me + N_DEV - 1, N_DEV)
        s_val = s_ref[0, 0]

        barrier = pltpu.get_barrier_semaphore()
        for d in range(1, N_DEV):
            peer = lax.rem(me + d, N_DEV)
            pl.semaphore_signal(
                barrier, inc=1,
                device_id=(peer,), device_id_type=pl.DeviceIdType.MESH,
            )
        pl.semaphore_wait(barrier, N_DEV - 1)

        def w_send(buf, src_slot, dst_slot, p, ssems, rsems, dev):
            src = (w_ref.at[:, pl.ds(p * n_piece, n_piece)] if src_slot == 0
                   else buf.at[src_slot - 1, :, pl.ds(p * n_piece, n_piece)])
            return pltpu.make_async_remote_copy(
                src_ref=src,
                dst_ref=buf.at[dst_slot - 1, :, pl.ds(p * n_piece, n_piece)],
                send_sem=ssems.at[src_slot, p],
                recv_sem=rsems.at[dst_slot, p],
                device_id=(dev,),
                device_id_type=pl.DeviceIdType.MESH,
            )

        rdmas = []

        def start(rd):
            rd.start()
            rdmas.append(rd)

        start(w_send(wr_ref, 0, 1, 0, wr_send_sems, wr_recv_sems, right))
        start(w_send(wl_ref, 0, 1, 1, wl_send_sems, wl_recv_sems, left))
        start(w_send(wr_ref, 0, 1, 1, wr_send_sems, wr_recv_sems, right))
        start(w_send(wl_ref, 0, 1, 0, wl_send_sems, wl_recv_sems, left))

        for d in range(1, N_DEV):
            dest = lax.rem(me + d, N_DEV)
            start(pltpu.make_async_remote_copy(
                src_ref=x_ref.at[pl.ds(dest * m_per, m_per), :],
                dst_ref=xp_ref.at[d],
                send_sem=x_send_sems.at[d],
                recv_sem=x_recv_sems.at[d],
                device_id=(dest,),
                device_id_type=pl.DeviceIdType.MESH,
            ))

        xo = x_ref[pl.ds(me * m_per, m_per), :]
        out_ref[...] = dotf(xo, w_ref[...]) * s_val

        def x_wait(slot):
            pltpu.make_async_remote_copy(
                src_ref=xp_ref.at[slot],
                dst_ref=xp_ref.at[slot],
                send_sem=x_send_sems.at[slot],
                recv_sem=x_recv_sems.at[slot],
                device_id=(right,),
                device_id_type=pl.DeviceIdType.MESH,
            ).wait_recv()

        def stream_step(buf, st, pieces, nxt_pieces, ssems, rsems, dev,
                        xp_slot, wait_x=True):
            for p in pieces:
                w_send(buf, st, st, p, ssems, rsems, dev).wait_recv()
                if st < N_HOP and p in nxt_pieces:
                    start(w_send(buf, st, st + 1, p, ssems, rsems, dev))
                if wait_x and p == pieces[0]:
                    x_wait(xp_slot)
                cols = pl.ds(p * n_piece, n_piece)
                out_ref[:, cols] = out_ref[:, cols] + dotf(
                    xp_ref[xp_slot], buf[st - 1, :, cols]
                ) * s_val

        def r_pieces(st):
            return [0] if st == N_HOP else [0, 1]

        def l_pieces(st):
            return [1] if st == N_HOP else [1, 0]

        for st in range(1, N_HOP + 1):
            stream_step(wr_ref, st, r_pieces(st), r_pieces(st + 1),
                        wr_send_sems, wr_recv_sems, right, st)
            stream_step(wl_ref, st, l_pieces(st), l_pieces(st + 1),
                        wl_send_sems, wl_recv_sems, left, N_DEV - st,
                        wait_x=(st < N_HOP))

        for rd in rdmas:
            rd.wait_send()

    return pl.pallas_call(
        body,
        out_shape=jax.ShapeDtypeStruct((m_per, N), jnp.float32),
        in_specs=[
            pl.BlockSpec(memory_space=pltpu.VMEM),
            pl.BlockSpec(memory_space=pltpu.VMEM),
            pl.BlockSpec(memory_space=pltpu.SMEM),
        ],
        out_specs=pl.BlockSpec(memory_space=pltpu.VMEM),
        scratch_shapes=[
            pltpu.VMEM((N_DEV, m_per, Ks), jnp.float8_e4m3fn),
            pltpu.VMEM((N_HOP, Ks, N), jnp.float8_e4m3fn),
            pltpu.VMEM((N_HOP, Ks, N), jnp.float8_e4m3fn),
            pltpu.SemaphoreType.DMA((N_DEV,)),
            pltpu.SemaphoreType.DMA((N_DEV,)),
            pltpu.SemaphoreType.DMA((N_HOP + 1, NPC)),
            pltpu.SemaphoreType.DMA((N_HOP + 1, NPC)),
            pltpu.SemaphoreType.DMA((N_HOP + 1, NPC)),
            pltpu.SemaphoreType.DMA((N_HOP + 1, NPC)),
        ],
        compiler_params=pltpu.CompilerParams(
            collective_id=0,
            vmem_limit_bytes=100 * 1024 * 1024,
        ),
    )(x8, w8, s)


# device time: 215318 ns/iter; 1.7747x vs baseline; 1.0057x over previous
import jax
import jax.numpy as jnp
from jax import lax
from jax.experimental import pallas as pl
from jax.experimental.pallas import tpu as pltpu

N_DEV = 8
N_HOP = 4
NPC = 4


def kernel(x, w_mat, scale_x, scale_w):
    M, Ks = x.shape
    N = w_mat.shape[1]
    m_per = M // N_DEV
    n_piece = N // NPC

    x8 = x.astype(jnp.float8_e4m3fn)
    w8 = w_mat.astype(jnp.float8_e4m3fn)
    s = (scale_x * scale_w).reshape(1, 1)

    def dotf(a, b):
        return lax.dot_general(
            a, b, (((1,), (0,)), ((), ())),
            preferred_element_type=jnp.float32,
        )

    def body(x_ref, w_ref, s_ref, out_ref, xp_ref, wr_ref, wl_ref,
             x_send_sems, x_recv_sems,
             wr_send_sems, wr_recv_sems, wl_send_sems, wl_recv_sems):
        me = lax.axis_index("i")
        right = lax.rem(me + 1, N_DEV)
        left = lax.rem(me + N_DEV - 1, N_DEV)
        s_val = s_ref[0, 0]

        barrier = pltpu.get_barrier_semaphore()
        for d in range(1, N_DEV):
            peer = lax.rem(me + d, N_DEV)
            pl.semaphore_signal(
                barrier, inc=1,
                device_id=(peer,), device_id_type=pl.DeviceIdType.MESH,
            )
        pl.semaphore_wait(barrier, N_DEV - 1)

        def w_send(buf, src_slot, dst_slot, p, ssems, rsems, dev):
            src = (w_ref.at[:, pl.ds(p * n_piece, n_piece)] if src_slot == 0
                   else buf.at[src_slot - 1, :, pl.ds(p * n_piece, n_piece)])
            return pltpu.make_async_remote_copy(
                src_ref=src,
                dst_ref=buf.at[dst_slot - 1, :, pl.ds(p * n_piece, n_piece)],
                send_sem=ssems.at[src_slot, p],
                recv_sem=rsems.at[dst_slot, p],
                device_id=(dev,),
                device_id_type=pl.DeviceIdType.MESH,
            )

        rdmas = []

        def start(rd):
            rd.start()
            rdmas.append(rd)

        def r_pieces(st):
            return list(range(NPC // 2)) if st == N_HOP else list(range(NPC))

        def l_pieces(st):
            lead = list(range(NPC // 2, NPC))
            return lead if st == N_HOP else lead + list(range(NPC // 2))

        for pr, pll in zip(r_pieces(1), l_pieces(1)):
            start(w_send(wr_ref, 0, 1, pr, wr_send_sems, wr_recv_sems,
                         right))
            start(w_send(wl_ref, 0, 1, pll, wl_send_sems, wl_recv_sems,
                         left))

        for d in range(1, N_DEV):
            dest = lax.rem(me + d, N_DEV)
            start(pltpu.make_async_remote_copy(
                src_ref=x_ref.at[pl.ds(dest * m_per, m_per), :],
                dst_ref=xp_ref.at[d],
                send_sem=x_send_sems.at[d],
                recv_sem=x_recv_sems.at[d],
                device_id=(dest,),
                device_id_type=pl.DeviceIdType.MESH,
            ))

        xo = x_ref[pl.ds(me * m_per, m_per), :]
        out_ref[...] = dotf(xo, w_ref[...]) * s_val

        def x_wait(slot):
            pltpu.make_async_remote_copy(
                src_ref=xp_ref.at[slot],
                dst_ref=xp_ref.at[slot],
                send_sem=x_send_sems.at[slot],
                recv_sem=x_recv_sems.at[slot],
                device_id=(right,),
                device_id_type=pl.DeviceIdType.MESH,
            ).wait_recv()

        def stream_step(buf, st, pieces, nxt_pieces, ssems, rsems, dev,
                        xp_slot, wait_x=True):
            for p in pieces:
                w_send(buf, st, st, p, ssems, rsems, dev).wait_recv()
                if st < N_HOP and p in nxt_pieces:
                    start(w_send(buf, st, st + 1, p, ssems, rsems, dev))
                if wait_x and p == pieces[0]:
                    x_wait(xp_slot)
                cols = pl.ds(p * n_piece, n_piece)
                out_ref[:, cols] = out_ref[:, cols] + dotf(
                    xp_ref[xp_slot], buf[st - 1, :, cols]
                ) * s_val

        for st in range(1, N_HOP + 1):
            stream_step(wr_ref, st, r_pieces(st), r_pieces(st + 1),
                        wr_send_sems, wr_recv_sems, right, st)
            stream_step(wl_ref, st, l_pieces(st), l_pieces(st + 1),
                        wl_send_sems, wl_recv_sems, left, N_DEV - st,
                        wait_x=(st < N_HOP))

        for rd in rdmas:
            rd.wait_send()

    return pl.pallas_call(
        body,
        out_shape=jax.ShapeDtypeStruct((m_per, N), jnp.float32),
        in_specs=[
            pl.BlockSpec(memory_space=pltpu.VMEM),
            pl.BlockSpec(memory_space=pltpu.VMEM),
            pl.BlockSpec(memory_space=pltpu.SMEM),
        ],
        out_specs=pl.BlockSpec(memory_space=pltpu.VMEM),
        scratch_shapes=[
            pltpu.VMEM((N_DEV, m_per, Ks), jnp.float8_e4m3fn),
            pltpu.VMEM((N_HOP, Ks, N), jnp.float8_e4m3fn),
            pltpu.VMEM((N_HOP, Ks, N), jnp.float8_e4m3fn),
            pltpu.SemaphoreType.DMA((N_DEV,)),
            pltpu.SemaphoreType.DMA((N_DEV,)),
            pltpu.SemaphoreType.DMA((N_HOP + 1, NPC)),
            pltpu.SemaphoreType.DMA((N_HOP + 1, NPC)),
            pltpu.SemaphoreType.DMA((N_HOP + 1, NPC)),
            pltpu.SemaphoreType.DMA((N_HOP + 1, NPC)),
        ],
        compiler_params=pltpu.CompilerParams(
            collective_id=0,
            vmem_limit_bytes=100 * 1024 * 1024,
        ),
    )(x8, w8, s)
